# baseline (speedup 1.0000x reference)
import functools

import jax
import jax.numpy as jnp
from jax import lax
from jax.experimental import pallas as pl
from jax.experimental.pallas import tpu as pltpu
from jax.experimental.pallas import tpu_sc as plsc


def _pos_embed_sc(batch, seq, d):
    info = plsc.get_sparse_core_info()
    nc, ns = info.num_cores, info.num_subcores
    nw = nc * ns
    rows = seq // nw
    mesh = plsc.VectorSubcoreMesh(core_axis_name="c", subcore_axis_name="s")

    @functools.partial(
        pl.kernel,
        out_type=jax.ShapeDtypeStruct((batch, seq, d), jnp.float32),
        mesh=mesh,
        scratch_types=[
            pltpu.VMEM((rows, d), jnp.float32),
            pltpu.SemaphoreType.DMA,
        ],
    )
    def k(w_hbm, out_hbm, rows_v, sem):
        wid = lax.axis_index("s") * nc + lax.axis_index("c")
        base = wid * rows
        pltpu.sync_copy(w_hbm.at[pl.ds(base, rows)], rows_v)
        cps = []
        for b in range(batch):
            cp = pltpu.make_async_copy(
                rows_v, out_hbm.at[b, pl.ds(base, rows)], sem)
            cp.start()
            cps.append(cp)
        for cp in cps:
            cp.wait()

    return k


def kernel(tokens, W_pos):
    batch, seq = tokens.shape
    d = W_pos.shape[-1]
    return _pos_embed_sc(batch, seq, d)(W_pos[:seq])
